# baseline (speedup 1.0000x reference)
import functools

import jax
import jax.numpy as jnp
from jax import lax
from jax.experimental import pallas as pl
from jax.experimental.pallas import tpu as pltpu
from jax.experimental.pallas import tpu_sc as plsc

_NU = 50000
_NE = 50000
_D = 64
_NREL = 32
_HALF = 25000
_TPS = 1568
_ACC_ROWS = 25088
_DUMP = 25000
_CHK = 256
_CHU = 128
_Q = 512
_NK = 800000
_NNZ = 1000000
_SLK = 50176
_SLU = 63488
_LK = _NK + 16 * _Q * 2 + 16
_LU = _NNZ + 16 * _Q * 2 + 16
_FBLK = 2000


def _segs(total, ch):
    segs, o = [], 0
    while o < total:
        n = min(ch, total - o)
        segs.append((o, n))
        o += n
    return tuple(segs)


def _norm_rows(num, den):
    agg = num / (den + 1e-16)
    n = jnp.sqrt(jnp.sum(agg * agg, axis=1, keepdims=True))
    return agg / jnp.maximum(n, 1e-12)


def _w0_body(e_ref, relt_ref, w_ref):
    w_ref[...] = jnp.dot(e_ref[...], relt_ref[...],
                         preferred_element_type=jnp.float32)


@jax.jit
def _w0(e, relt):
    grid = (_NE // _FBLK,)
    return pl.pallas_call(
        _w0_body,
        grid=grid,
        in_specs=[pl.BlockSpec((_FBLK, _D), lambda i: (i, 0)),
                  pl.BlockSpec((_D, _NREL), lambda i: (0, 0))],
        out_specs=pl.BlockSpec((_FBLK, _NREL), lambda i: (i, 0)),
        out_shape=jax.ShapeDtypeStruct((_NE, _NREL), jnp.float32),
    )(e, relt)


def _finish_body(num_e_ref, den_e_ref, num_u_ref, den_u_ref,
                 eres_ref, ures_ref, relt_ref,
                 enew_ref, unew_ref, eout_ref, uout_ref, w_ref):
    e = _norm_rows(num_e_ref[...], den_e_ref[...])
    u = _norm_rows(num_u_ref[...], den_u_ref[...])
    enew_ref[...] = e
    unew_ref[...] = u
    eout_ref[...] = eres_ref[...] + e
    uout_ref[...] = ures_ref[...] + u
    w_ref[...] = jnp.dot(e, relt_ref[...], preferred_element_type=jnp.float32)


@jax.jit
def _finish(num_e, den_e, num_u, den_u, eres, ures, relt):
    grid = (_NE // _FBLK,)
    row_spec = pl.BlockSpec((_FBLK, _D), lambda i: (i, 0))
    col_spec = pl.BlockSpec((_FBLK, 1), lambda i: (i, 0))
    return pl.pallas_call(
        _finish_body,
        grid=grid,
        in_specs=[row_spec, col_spec, row_spec, col_spec, row_spec, row_spec,
                  pl.BlockSpec((_D, _NREL), lambda i: (0, 0))],
        out_specs=[row_spec, row_spec, row_spec, row_spec,
                   pl.BlockSpec((_FBLK, _NREL), lambda i: (i, 0))],
        out_shape=[jax.ShapeDtypeStruct((_NE, _D), jnp.float32)] * 4
        + [jax.ShapeDtypeStruct((_NE, _NREL), jnp.float32)],
    )(num_e, den_e[:, None], num_u, den_u[:, None], eres, ures, relt)


_mesh = plsc.VectorSubcoreMesh(core_axis_name="c", subcore_axis_name="s")
_IOTA = functools.partial(lax.broadcasted_iota, jnp.int32, (16,), 0)


def _tile_ranges(meta_v, cid, sid, ch):
    m = meta_v[pl.ds(0, 16)]
    pt_edges = jnp.where(cid == 0, m[0], m[1])
    base_edge = jnp.where(cid == 0, 0, m[2])
    start_edge = pl.multiple_of(base_edge + sid * pt_edges, ch)
    nch = pt_edges // ch
    return start_edge, nch


def _zero_acc(accn, accd, z2_hbm, z1_hbm, rows_v, p_v, sid, ch):
    pltpu.sync_copy(z2_hbm.at[pl.ds(0, ch)], rows_v)
    pltpu.sync_copy(z1_hbm.at[pl.ds(0, ch)], p_v)
    row0 = sid * _TPS
    for o, n in _segs(_TPS, ch):
        pltpu.sync_copy(rows_v.at[pl.ds(0, n)],
                        accn.at[pl.ds(row0 + o, n)])
        pltpu.sync_copy(p_v.at[pl.ds(0, n)],
                        accd.at[pl.ds(row0 + o, n)])
    plsc.subcore_barrier()


def _writeout(accn, accd, num_hbm, den_hbm, rows_v, p_v, cid, sid, ch):
    plsc.subcore_barrier()
    row0 = sid * _TPS
    gbase = cid * _HALF + row0

    def flush(segs):
        for o, n in segs:
            pltpu.sync_copy(accn.at[pl.ds(row0 + o, n)],
                            rows_v.at[pl.ds(0, n)])
            pltpu.sync_copy(rows_v.at[pl.ds(0, n)],
                            num_hbm.at[pl.ds(gbase + o, n)])
            pltpu.sync_copy(accd.at[pl.ds(row0 + o, n)],
                            p_v.at[pl.ds(0, n)])
            pltpu.sync_copy(p_v.at[pl.ds(0, n)],
                            den_hbm.at[pl.ds(gbase + o, n)])

    @pl.when(sid < 15)
    def _():
        flush(_segs(_TPS, ch))

    @pl.when(sid == 15)
    def _():
        flush(_segs(_HALF - 15 * _TPS, ch))


@functools.partial(
    pl.kernel,
    out_type=[jax.ShapeDtypeStruct((_NE, _D), jnp.float32),
              jax.ShapeDtypeStruct((_NE,), jnp.float32)],
    mesh=_mesh,
    scratch_types=[
        pltpu.VMEM_SHARED((_ACC_ROWS, _D), jnp.float32),
        pltpu.VMEM_SHARED((_ACC_ROWS,), jnp.float32),
        pltpu.VMEM((_CHK,), jnp.int32),
        pltpu.VMEM((_CHK,), jnp.int32),
        pltpu.VMEM((2, 128), jnp.int32),
        pltpu.VMEM((_CHK, _D), jnp.float32),
        pltpu.VMEM((_CHK,), jnp.float32),
        pltpu.VMEM((_CHK,), jnp.float32),
        pltpu.VMEM((16,), jnp.int32),
        pltpu.SemaphoreType.DMA,
        pltpu.SemaphoreType.DMA,
    ],
    compiler_params=pltpu.CompilerParams(
        needs_layout_passes=False, use_tc_tiling_on_sc=False),
)
def _sc_kg(ent_hbm, wflat_hbm, tail_hbm, widx_hbm, hloc_hbm, meta_hbm,
           z2_hbm, z1_hbm, num_hbm, den_hbm,
           accn, accd, tidx_v, widx_v, hloc_v, rows_v, w_v, p_v, meta_v,
           gsem, ssem):
    cid = lax.axis_index("c")
    sid = lax.axis_index("s")
    pltpu.sync_copy(meta_hbm, meta_v)
    start_edge, nch = _tile_ranges(meta_v, cid, sid, _CHK)
    _zero_acc(accn, accd, z2_hbm, z1_hbm, rows_v, p_v, sid, _CHK)

    iota = _IOTA()

    def chunk(i, _):
        eoff = pl.multiple_of(start_edge + i * _CHK, _CHK)
        pltpu.sync_copy(tail_hbm.at[pl.ds(eoff, _CHK)], tidx_v)
        pltpu.sync_copy(widx_hbm.at[pl.ds(eoff, _CHK)], widx_v)
        for j in range(2):
            pltpu.sync_copy(hloc_hbm.at[pl.ds(eoff + j * 128, 128)],
                            hloc_v.at[j])
        descs = []
        for j in range(2):
            descs.append(pltpu.async_copy(
                ent_hbm.at[tidx_v.at[pl.ds(j * 128, 128)]],
                rows_v.at[pl.ds(j * 128, 128)], gsem))
            descs.append(pltpu.async_copy(
                wflat_hbm.at[widx_v.at[pl.ds(j * 128, 128)]],
                w_v.at[pl.ds(j * 128, 128)], gsem))
        for d in descs:
            d.wait()

        def g_body(g, _):
            w16 = w_v[pl.ds(g * 16, 16)]
            p16 = jnp.exp(w16)
            p_v[pl.ds(g * 16, 16)] = p16
            rbase = iota + g * 16

            def c_body(c8, _):
                for k in range(8):
                    c = c8 * 8 + k
                    col = jnp.full((16,), c, jnp.int32)
                    v = plsc.load_gather(rows_v, [rbase, col])
                    plsc.store_scatter(rows_v, [rbase, col], v * p16)
                return 0

            lax.fori_loop(0, 8, c_body, 0)
            return 0

        lax.fori_loop(0, _CHK // 16, g_body, 0)

        descs = []
        for j in range(2):
            descs.append(pltpu.async_copy(
                rows_v.at[pl.ds(j * 128, 128)],
                accn.at[hloc_v.at[j]], ssem, add=True))
            descs.append(pltpu.async_copy(
                p_v.at[pl.ds(j * 128, 128)],
                accd.at[hloc_v.at[j]], ssem, add=True))
        for d in descs:
            d.wait()
        return 0

    lax.fori_loop(0, nch, chunk, 0)
    _writeout(accn, accd, num_hbm, den_hbm, rows_v, p_v, cid, sid, _CHK)


@functools.partial(
    pl.kernel,
    out_type=[jax.ShapeDtypeStruct((_NU, _D), jnp.float32),
              jax.ShapeDtypeStruct((_NU,), jnp.float32)],
    mesh=_mesh,
    scratch_types=[
        pltpu.VMEM_SHARED((_ACC_ROWS, _D), jnp.float32),
        pltpu.VMEM_SHARED((_ACC_ROWS,), jnp.float32),
        pltpu.VMEM((_CHU,), jnp.int32),
        pltpu.VMEM((_CHU,), jnp.int32),
        pltpu.VMEM((1, 128), jnp.int32),
        pltpu.VMEM((_CHU, _D), jnp.float32),
        pltpu.VMEM((_CHU, _D), jnp.float32),
        pltpu.VMEM((_CHU,), jnp.int32),
        pltpu.VMEM((_CHU,), jnp.float32),
        pltpu.VMEM((256,), jnp.float32),
        pltpu.VMEM((16,), jnp.int32),
        pltpu.SemaphoreType.DMA,
        pltpu.SemaphoreType.DMA,
    ],
    compiler_params=pltpu.CompilerParams(
        needs_layout_passes=False, use_tc_tiling_on_sc=False),
)
def _sc_user(ent_hbm, utbl_hbm, te_hbm, item_hbm, usr_hbm, typ_hbm, uloc_hbm,
             meta_hbm, z2_hbm, z1_hbm, num_hbm, den_hbm,
             accn, accd, iidx_v, uidx_v, uloc_v, ie_v, ue_v, typ_v, p_v,
             te_v, meta_v, gsem, ssem):
    cid = lax.axis_index("c")
    sid = lax.axis_index("s")
    pltpu.sync_copy(meta_hbm, meta_v)
    pltpu.sync_copy(te_hbm, te_v)
    start_edge, nch = _tile_ranges(meta_v, cid, sid, _CHU)
    _zero_acc(accn, accd, z2_hbm, z1_hbm, ie_v, p_v, sid, _CHU)

    iota = _IOTA()

    def chunk(i, _):
        eoff = pl.multiple_of(start_edge + i * _CHU, _CHU)
        pltpu.sync_copy(item_hbm.at[pl.ds(eoff, _CHU)], iidx_v)
        pltpu.sync_copy(usr_hbm.at[pl.ds(eoff, _CHU)], uidx_v)
        pltpu.sync_copy(typ_hbm.at[pl.ds(eoff, _CHU)], typ_v)
        pltpu.sync_copy(uloc_hbm.at[pl.ds(eoff, 128)], uloc_v.at[0])
        descs = [
            pltpu.async_copy(ent_hbm.at[iidx_v], ie_v, gsem),
            pltpu.async_copy(utbl_hbm.at[uidx_v], ue_v, gsem),
        ]
        for d in descs:
            d.wait()

        def g_body(g, _):
            rbase = iota + g * 16
            tb = typ_v[pl.ds(g * 16, 16)] * 64

            def att_body(c8, att):
                for k in range(8):
                    c = c8 * 8 + k
                    col = jnp.full((16,), c, jnp.int32)
                    ue = plsc.load_gather(ue_v, [rbase, col])
                    ie = plsc.load_gather(ie_v, [rbase, col])
                    te = plsc.load_gather(te_v, [tb + c])
                    att = att + ue * te * ie
                return att

            att = lax.fori_loop(0, 8, att_body,
                                jnp.zeros((16,), jnp.float32))
            p16 = jnp.exp(att)
            p_v[pl.ds(g * 16, 16)] = p16

            def sc_body(c8, _):
                for k in range(8):
                    c = c8 * 8 + k
                    col = jnp.full((16,), c, jnp.int32)
                    ie = plsc.load_gather(ie_v, [rbase, col])
                    plsc.store_scatter(ie_v, [rbase, col], ie * p16)
                return 0

            lax.fori_loop(0, 8, sc_body, 0)
            return 0

        lax.fori_loop(0, _CHU // 16, g_body, 0)

        descs = [
            pltpu.async_copy(ie_v, accn.at[uloc_v.at[0]], ssem, add=True),
            pltpu.async_copy(p_v, accd.at[uloc_v.at[0]], ssem, add=True),
        ]
        for d in descs:
            d.wait()
        return 0

    lax.fori_loop(0, nch, chunk, 0)
    _writeout(accn, accd, num_hbm, den_hbm, ie_v, p_v, cid, sid, _CHU)


_QP = 1024


def _make_count(sl):
    @functools.partial(
        pl.kernel,
        out_type=jax.ShapeDtypeStruct((512,), jnp.int32),
        mesh=_mesh,
        scratch_types=[
            pltpu.VMEM((_QP,), jnp.int32),
            pltpu.VMEM((16,), jnp.int32),
        ],
        compiler_params=pltpu.CompilerParams(
            needs_layout_passes=False, use_tc_tiling_on_sc=False),
    )
    def k(dst_hbm, cnt_hbm, dst_v, out_v):
        cid = lax.axis_index("c")
        sid = lax.axis_index("s")
        wid = cid * 16 + sid
        lo = cid * _HALF

        def cb(i, tot):
            eo = pl.multiple_of(sid * sl + i * _QP, _QP)
            pltpu.sync_copy(dst_hbm.at[pl.ds(eo, _QP)], dst_v)

            def g(gi, t):
                d = dst_v[pl.ds(gi * 16, 16)]
                msk = jnp.logical_and(d >= lo, d < lo + _HALF)
                return t + plsc.all_reduce_population_count(msk)

            return lax.fori_loop(0, _QP // 16, g, tot)

        tot = lax.fori_loop(0, sl // _QP, cb, jnp.zeros((16,), jnp.int32))
        out_v[pl.ds(0, 16)] = tot
        pltpu.sync_copy(out_v,
                        cnt_hbm.at[pl.ds(pl.multiple_of(wid * 16, 16), 16)])

    return k


def _glue(cnt):
    counts = cnt.reshape(32, 16)[:, 0]
    c0, c1 = counts[:16], counts[16:]
    n0 = jnp.sum(c0)
    n1 = jnp.sum(c1)
    q16 = 16 * _Q
    pt0 = ((n0 + q16 - 1) // q16) * _Q
    pt1 = ((n1 + q16 - 1) // q16) * _Q
    b1 = 16 * pt0
    off = jnp.concatenate([jnp.cumsum(c0) - c0, b1 + jnp.cumsum(c1) - c1])
    a32 = jnp.arange(32, dtype=jnp.int32)
    ptc = jnp.where(a32 < 16, pt0, pt1)
    basec = jnp.where(a32 < 16, 0, b1)
    z32 = jnp.zeros((32,), jnp.int32)
    tbl = jnp.stack([off, ptc, basec] + [z32] * 13, axis=1).reshape(-1)
    meta = jnp.stack([pt0, pt1, b1] + [jnp.int32(0)] * 13)
    return meta.astype(jnp.int32), tbl.astype(jnp.int32)


def _make_place(sl, L, kind):
    npay = 3 if kind == "kg" else 4
    nj = _QP // 128

    @functools.partial(
        pl.kernel,
        out_type=[jax.ShapeDtypeStruct((L,), jnp.int32)] * npay,
        mesh=_mesh,
        scratch_types=[
            pltpu.VMEM((_QP,), jnp.int32),
            pltpu.VMEM((_QP,), jnp.int32),
            pltpu.VMEM((_QP,), jnp.int32),
            pltpu.VMEM((_QP,), jnp.int32),
            pltpu.VMEM((_QP,), jnp.int32),
            pltpu.VMEM((nj, 128), jnp.int32),
            pltpu.VMEM((16,), jnp.int32),
            pltpu.SemaphoreType.DMA,
            pltpu.SemaphoreType.DMA,
            pltpu.SemaphoreType.DMA,
        ],
        compiler_params=pltpu.CompilerParams(
            needs_layout_passes=False, use_tc_tiling_on_sc=False),
    )
    def k(dst_hbm, a_hbm, b_hbm, tbl_hbm, *rest):
        outs = rest[:npay]
        (dst_v, a_v, b_v, st1, st2, slots_v, off_v,
         fsem, gsem, ssem) = rest[npay:]
        cid = lax.axis_index("c")
        sid = lax.axis_index("s")
        wid = cid * 16 + sid
        lo = cid * _HALF
        iota = _IOTA()
        pltpu.sync_copy(tbl_hbm.at[pl.ds(pl.multiple_of(wid * 16, 16), 16)],
                        off_v)
        m = off_v[pl.ds(0, 16)]
        off0, pt, base = m[0], m[1], m[2]

        def sb(g, _):
            dst_v[pl.ds(g * 16, 16)] = jnp.zeros((16,), jnp.int32)
            a_v[pl.ds(g * 16, 16)] = jnp.full((16,), _DUMP, jnp.int32)
            return 0

        lax.fori_loop(0, _QP // 16, sb, 0)
        sent = [dst_v] * (npay - 1) + [a_v]

        def fill_issue(o, n):
            for p in range(npay):
                pltpu.async_copy(sent[p].at[pl.ds(0, n)],
                                 outs[p].at[pl.ds(o, n)], fsem)

        def fill_drain(n):
            for p in range(npay):
                pltpu.make_async_copy(outs[p].at[pl.ds(0, n)],
                                      b_v.at[pl.ds(0, n)], fsem).wait()

        nf = pt // _QP

        def fb(i, _):
            o = pl.multiple_of(base + sid * pt + i * _QP, _Q)
            fill_issue(o, _QP)

            @pl.when(i > 0)
            def _():
                fill_drain(_QP)

            return 0

        lax.fori_loop(0, nf, fb, 0)

        @pl.when(nf > 0)
        def _():
            fill_drain(_QP)

        @pl.when(pt - nf * _QP > 0)
        def _():
            o = pl.multiple_of(base + sid * pt + nf * _QP, _Q)
            fill_issue(o, _Q)
            fill_drain(_Q)

        plsc.subcore_barrier()

        trash = L - 16 + cid * 8

        def data_list():
            if kind == "kg":
                return [a_v, st1, st2]
            return [a_v, dst_v, b_v, st1]

        def scb(i, off):
            @pl.when(i > 0)
            def _():
                for p in range(npay):
                    for j in range(nj):
                        pltpu.make_async_copy(outs[p].at[pl.ds(0, 128)],
                                              slots_v.at[0], ssem).wait()

            eo = pl.multiple_of(sid * sl + i * _QP, _QP)
            d1 = pltpu.async_copy(dst_hbm.at[pl.ds(eo, _QP)], dst_v, gsem)
            d2 = pltpu.async_copy(a_hbm.at[pl.ds(eo, _QP)], a_v, gsem)
            d3 = pltpu.async_copy(b_hbm.at[pl.ds(eo, _QP)], b_v, gsem)
            d1.wait()
            d2.wait()
            d3.wait()

            def g_body(g, off):
                d16 = dst_v[pl.ds(g * 16, 16)]
                msk = jnp.logical_and(d16 >= lo, d16 < lo + _HALF)
                r = plsc.cumsum(msk.astype(jnp.int32))
                pop = r[15]
                slot = jnp.where(msk, off + r - 1,
                                 trash + jnp.bitwise_and(iota, 7))
                slots_v[g // 8, pl.ds((g % 8) * 16, 16)] = slot
                if kind == "kg":
                    a16 = a_v[pl.ds(g * 16, 16)]
                    b16 = b_v[pl.ds(g * 16, 16)]
                    st1[pl.ds(g * 16, 16)] = a16 * _NREL + b16 - 1
                    st2[pl.ds(g * 16, 16)] = d16 - lo
                else:
                    st1[pl.ds(g * 16, 16)] = d16 - lo
                return off + pop

            off = lax.fori_loop(0, _QP // 16, g_body, off)
            dl = data_list()
            for p in range(npay):
                for j in range(nj):
                    pltpu.async_copy(dl[p].at[pl.ds(j * 128, 128)],
                                     outs[p].at[slots_v.at[j]], ssem)
            return off

        lax.fori_loop(0, sl // _QP, scb, off0)
        for p in range(npay):
            for j in range(nj):
                pltpu.make_async_copy(outs[p].at[pl.ds(0, 128)],
                                      slots_v.at[0], ssem).wait()

    return k


_count_kg = _make_count(_SLK)
_count_u = _make_count(_SLU)
_place_kg = _make_place(_SLK, _LK, "kg")
_place_u = _make_place(_SLU, _LU, "user")


def kernel(user_emb, entity_emb, interact_emb, relation_emb, edge_index,
           edge_type, interact_user_index, interact_item_index, interact_type):
    f32 = jnp.float32
    head = edge_index[0].astype(jnp.int32)
    tail = edge_index[1].astype(jnp.int32)
    etype = edge_type.astype(jnp.int32)
    uidx = interact_user_index.astype(jnp.int32)
    iidx = interact_item_index.astype(jnp.int32)
    titype = interact_type.astype(jnp.int32)

    def pad_to(x, n, v):
        return jnp.concatenate(
            [x, jnp.full((n - x.shape[0],), v, jnp.int32)])

    head_pad = pad_to(head, 16 * _SLK, 1 << 29)
    tail_pad = pad_to(tail, 16 * _SLK, 0)
    etype_pad = pad_to(etype, 16 * _SLK, 1)
    usr_pad = pad_to(uidx, 16 * _SLU, 1 << 29)
    item_pad = pad_to(iidx, 16 * _SLU, 0)
    typ_pad = pad_to(titype, 16 * _SLU, 0)

    meta_k, tbl_k = _glue(_count_kg(head_pad))
    meta_u, tbl_u = _glue(_count_u(usr_pad))
    tail_p, widx_p, hloc_p = _place_kg(head_pad, tail_pad, etype_pad, tbl_k)
    item_p, usr_p, typ_p, uloc_p = _place_u(usr_pad, item_pad, typ_pad,
                                            tbl_u)

    z2 = jnp.zeros((_CHK, _D), f32)
    z1 = jnp.zeros((_CHK,), f32)
    relt = relation_emb.T.astype(f32) * 0.125
    te_flat = interact_emb.reshape(-1).astype(f32)

    e = entity_emb
    u = user_emb
    eres = entity_emb
    ures = user_emb
    w = _w0(e, relt)
    for _ in range(3):
        num_e, den_e = _sc_kg(e, w.reshape(-1), tail_p, widx_p, hloc_p,
                              meta_k, z2, z1)
        num_u, den_u = _sc_user(e, u, te_flat, item_p, usr_p, typ_p, uloc_p,
                                meta_u, z2, z1)
        e, u, eres, ures, w = _finish(num_e, den_e, num_u, den_u,
                                      eres, ures, relt)
    return (eres, ures)


# speedup vs baseline: 5.8757x; 5.8757x over previous
import functools

import jax
import jax.numpy as jnp
from jax import lax
from jax.experimental import pallas as pl
from jax.experimental.pallas import tpu as pltpu
from jax.experimental.pallas import tpu_sc as plsc

_NU = 50000
_NE = 50000
_D = 64
_NREL = 32
_HALF = 25000
_TPS = 1568
_ACC_ROWS = 25088
_DUMP = 25000
_CHK = 256
_CHU = 128
_Q = 512
_NK = 800000
_NNZ = 1000000
_SLK = 50176
_SLU = 63488
_TR = 1024
_LK = _NK + 16 * _Q * 2 + 32 * _TR
_LU = _NNZ + 16 * _Q * 2 + 32 * _TR
_FBLK = 2000


def _segs(total, ch):
    segs, o = [], 0
    while o < total:
        n = min(ch, total - o)
        segs.append((o, n))
        o += n
    return tuple(segs)


def _norm_rows(num, den):
    agg = num / (den + 1e-16)
    n = jnp.sqrt(jnp.sum(agg * agg, axis=1, keepdims=True))
    return agg / jnp.maximum(n, 1e-12)


def _w0_body(e_ref, relt_ref, w_ref):
    w_ref[...] = jnp.dot(e_ref[...], relt_ref[...],
                         preferred_element_type=jnp.float32)


@jax.jit
def _w0(e, relt):
    grid = (_NE // _FBLK,)
    return pl.pallas_call(
        _w0_body,
        grid=grid,
        in_specs=[pl.BlockSpec((_FBLK, _D), lambda i: (i, 0)),
                  pl.BlockSpec((_D, _NREL), lambda i: (0, 0))],
        out_specs=pl.BlockSpec((_FBLK, _NREL), lambda i: (i, 0)),
        out_shape=jax.ShapeDtypeStruct((_NE, _NREL), jnp.float32),
    )(e, relt)


def _finish_body(num_e_ref, den_e_ref, num_u_ref, den_u_ref,
                 eres_ref, ures_ref, relt_ref,
                 enew_ref, unew_ref, eout_ref, uout_ref, w_ref):
    e = _norm_rows(num_e_ref[...], den_e_ref[...])
    u = _norm_rows(num_u_ref[...], den_u_ref[...])
    enew_ref[...] = e
    unew_ref[...] = u
    eout_ref[...] = eres_ref[...] + e
    uout_ref[...] = ures_ref[...] + u
    w_ref[...] = jnp.dot(e, relt_ref[...], preferred_element_type=jnp.float32)


@jax.jit
def _finish(num_e, den_e, num_u, den_u, eres, ures, relt):
    grid = (_NE // _FBLK,)
    row_spec = pl.BlockSpec((_FBLK, _D), lambda i: (i, 0))
    col_spec = pl.BlockSpec((_FBLK, 1), lambda i: (i, 0))
    return pl.pallas_call(
        _finish_body,
        grid=grid,
        in_specs=[row_spec, col_spec, row_spec, col_spec, row_spec, row_spec,
                  pl.BlockSpec((_D, _NREL), lambda i: (0, 0))],
        out_specs=[row_spec, row_spec, row_spec, row_spec,
                   pl.BlockSpec((_FBLK, _NREL), lambda i: (i, 0))],
        out_shape=[jax.ShapeDtypeStruct((_NE, _D), jnp.float32)] * 4
        + [jax.ShapeDtypeStruct((_NE, _NREL), jnp.float32)],
    )(num_e, den_e[:, None], num_u, den_u[:, None], eres, ures, relt)


_mesh = plsc.VectorSubcoreMesh(core_axis_name="c", subcore_axis_name="s")
_IOTA = functools.partial(lax.broadcasted_iota, jnp.int32, (16,), 0)


def _tile_ranges(meta_v, cid, sid, ch):
    m = meta_v[pl.ds(0, 16)]
    pt_edges = jnp.where(cid == 0, m[0], m[1])
    base_edge = jnp.where(cid == 0, 0, m[2])
    start_edge = pl.multiple_of(base_edge + sid * pt_edges, ch)
    nch = pt_edges // ch
    return start_edge, nch


def _zero_acc(accn, accd, z2_hbm, z1_hbm, rows_v, p_v, sid, ch):
    pltpu.sync_copy(z2_hbm.at[pl.ds(0, ch)], rows_v)
    pltpu.sync_copy(z1_hbm.at[pl.ds(0, ch)], p_v)
    row0 = sid * _TPS
    for o, n in _segs(_TPS, ch):
        pltpu.sync_copy(rows_v.at[pl.ds(0, n)],
                        accn.at[pl.ds(row0 + o, n)])
        pltpu.sync_copy(p_v.at[pl.ds(0, n)],
                        accd.at[pl.ds(row0 + o, n)])
    plsc.subcore_barrier()


def _writeout(accn, accd, num_hbm, den_hbm, rows_v, p_v, cid, sid, ch):
    plsc.subcore_barrier()
    row0 = sid * _TPS
    gbase = cid * _HALF + row0

    def flush(segs):
        for o, n in segs:
            pltpu.sync_copy(accn.at[pl.ds(row0 + o, n)],
                            rows_v.at[pl.ds(0, n)])
            pltpu.sync_copy(rows_v.at[pl.ds(0, n)],
                            num_hbm.at[pl.ds(gbase + o, n)])
            pltpu.sync_copy(accd.at[pl.ds(row0 + o, n)],
                            p_v.at[pl.ds(0, n)])
            pltpu.sync_copy(p_v.at[pl.ds(0, n)],
                            den_hbm.at[pl.ds(gbase + o, n)])

    @pl.when(sid < 15)
    def _():
        flush(_segs(_TPS, ch))

    @pl.when(sid == 15)
    def _():
        flush(_segs(_HALF - 15 * _TPS, ch))


@functools.partial(
    pl.kernel,
    out_type=[jax.ShapeDtypeStruct((_NE, _D), jnp.float32),
              jax.ShapeDtypeStruct((_NE,), jnp.float32)],
    mesh=_mesh,
    scratch_types=[
        pltpu.VMEM_SHARED((_ACC_ROWS, _D), jnp.float32),
        pltpu.VMEM_SHARED((_ACC_ROWS,), jnp.float32),
        pltpu.VMEM((_CHK,), jnp.int32),
        pltpu.VMEM((_CHK,), jnp.int32),
        pltpu.VMEM((2, 128), jnp.int32),
        pltpu.VMEM((_CHK, _D), jnp.float32),
        pltpu.VMEM((_CHK,), jnp.float32),
        pltpu.VMEM((_CHK,), jnp.float32),
        pltpu.VMEM((16,), jnp.int32),
        pltpu.SemaphoreType.DMA,
        pltpu.SemaphoreType.DMA,
    ],
    compiler_params=pltpu.CompilerParams(
        needs_layout_passes=False, use_tc_tiling_on_sc=False),
)
def _sc_kg(ent_hbm, wflat_hbm, tail_hbm, widx_hbm, hloc_hbm, meta_hbm,
           z2_hbm, z1_hbm, num_hbm, den_hbm,
           accn, accd, tidx_v, widx_v, hloc_v, rows_v, w_v, p_v, meta_v,
           gsem, ssem):
    cid = lax.axis_index("c")
    sid = lax.axis_index("s")
    pltpu.sync_copy(meta_hbm, meta_v)
    start_edge, nch = _tile_ranges(meta_v, cid, sid, _CHK)
    _zero_acc(accn, accd, z2_hbm, z1_hbm, rows_v, p_v, sid, _CHK)

    iota = _IOTA()

    def chunk(i, _):
        eoff = pl.multiple_of(start_edge + i * _CHK, _CHK)
        pltpu.sync_copy(tail_hbm.at[pl.ds(eoff, _CHK)], tidx_v)
        pltpu.sync_copy(widx_hbm.at[pl.ds(eoff, _CHK)], widx_v)
        for j in range(2):
            pltpu.sync_copy(hloc_hbm.at[pl.ds(eoff + j * 128, 128)],
                            hloc_v.at[j])
        descs = []
        for j in range(2):
            descs.append(pltpu.async_copy(
                ent_hbm.at[tidx_v.at[pl.ds(j * 128, 128)]],
                rows_v.at[pl.ds(j * 128, 128)], gsem))
            descs.append(pltpu.async_copy(
                wflat_hbm.at[widx_v.at[pl.ds(j * 128, 128)]],
                w_v.at[pl.ds(j * 128, 128)], gsem))
        for d in descs:
            d.wait()

        def g_body(g, _):
            w16 = w_v[pl.ds(g * 16, 16)]
            p16 = jnp.exp(w16)
            p_v[pl.ds(g * 16, 16)] = p16
            rbase = iota + g * 16

            def c_body(c8, _):
                for k in range(8):
                    c = c8 * 8 + k
                    col = jnp.full((16,), c, jnp.int32)
                    v = plsc.load_gather(rows_v, [rbase, col])
                    plsc.store_scatter(rows_v, [rbase, col], v * p16)
                return 0

            lax.fori_loop(0, 8, c_body, 0)
            return 0

        lax.fori_loop(0, _CHK // 16, g_body, 0)

        descs = []
        for j in range(2):
            descs.append(pltpu.async_copy(
                rows_v.at[pl.ds(j * 128, 128)],
                accn.at[hloc_v.at[j]], ssem, add=True))
            descs.append(pltpu.async_copy(
                p_v.at[pl.ds(j * 128, 128)],
                accd.at[hloc_v.at[j]], ssem, add=True))
        for d in descs:
            d.wait()
        return 0

    lax.fori_loop(0, nch, chunk, 0)
    _writeout(accn, accd, num_hbm, den_hbm, rows_v, p_v, cid, sid, _CHK)


@functools.partial(
    pl.kernel,
    out_type=[jax.ShapeDtypeStruct((_NU, _D), jnp.float32),
              jax.ShapeDtypeStruct((_NU,), jnp.float32)],
    mesh=_mesh,
    scratch_types=[
        pltpu.VMEM_SHARED((_ACC_ROWS, _D), jnp.float32),
        pltpu.VMEM_SHARED((_ACC_ROWS,), jnp.float32),
        pltpu.VMEM((_CHU,), jnp.int32),
        pltpu.VMEM((_CHU,), jnp.int32),
        pltpu.VMEM((1, 128), jnp.int32),
        pltpu.VMEM((_CHU, _D), jnp.float32),
        pltpu.VMEM((_CHU, _D), jnp.float32),
        pltpu.VMEM((_CHU,), jnp.int32),
        pltpu.VMEM((_CHU,), jnp.float32),
        pltpu.VMEM((256,), jnp.float32),
        pltpu.VMEM((16,), jnp.int32),
        pltpu.SemaphoreType.DMA,
        pltpu.SemaphoreType.DMA,
    ],
    compiler_params=pltpu.CompilerParams(
        needs_layout_passes=False, use_tc_tiling_on_sc=False),
)
def _sc_user(ent_hbm, utbl_hbm, te_hbm, item_hbm, usr_hbm, typ_hbm, uloc_hbm,
             meta_hbm, z2_hbm, z1_hbm, num_hbm, den_hbm,
             accn, accd, iidx_v, uidx_v, uloc_v, ie_v, ue_v, typ_v, p_v,
             te_v, meta_v, gsem, ssem):
    cid = lax.axis_index("c")
    sid = lax.axis_index("s")
    pltpu.sync_copy(meta_hbm, meta_v)
    pltpu.sync_copy(te_hbm, te_v)
    start_edge, nch = _tile_ranges(meta_v, cid, sid, _CHU)
    _zero_acc(accn, accd, z2_hbm, z1_hbm, ie_v, p_v, sid, _CHU)

    iota = _IOTA()

    def chunk(i, _):
        eoff = pl.multiple_of(start_edge + i * _CHU, _CHU)
        pltpu.sync_copy(item_hbm.at[pl.ds(eoff, _CHU)], iidx_v)
        pltpu.sync_copy(usr_hbm.at[pl.ds(eoff, _CHU)], uidx_v)
        pltpu.sync_copy(typ_hbm.at[pl.ds(eoff, _CHU)], typ_v)
        pltpu.sync_copy(uloc_hbm.at[pl.ds(eoff, 128)], uloc_v.at[0])
        descs = [
            pltpu.async_copy(ent_hbm.at[iidx_v], ie_v, gsem),
            pltpu.async_copy(utbl_hbm.at[uidx_v], ue_v, gsem),
        ]
        for d in descs:
            d.wait()

        def g_body(g, _):
            rbase = iota + g * 16
            tb = typ_v[pl.ds(g * 16, 16)] * 64

            def att_body(c8, att):
                for k in range(8):
                    c = c8 * 8 + k
                    col = jnp.full((16,), c, jnp.int32)
                    ue = plsc.load_gather(ue_v, [rbase, col])
                    ie = plsc.load_gather(ie_v, [rbase, col])
                    te = plsc.load_gather(te_v, [tb + c])
                    att = att + ue * te * ie
                return att

            att = lax.fori_loop(0, 8, att_body,
                                jnp.zeros((16,), jnp.float32))
            p16 = jnp.exp(att)
            p_v[pl.ds(g * 16, 16)] = p16

            def sc_body(c8, _):
                for k in range(8):
                    c = c8 * 8 + k
                    col = jnp.full((16,), c, jnp.int32)
                    ie = plsc.load_gather(ie_v, [rbase, col])
                    plsc.store_scatter(ie_v, [rbase, col], ie * p16)
                return 0

            lax.fori_loop(0, 8, sc_body, 0)
            return 0

        lax.fori_loop(0, _CHU // 16, g_body, 0)

        descs = [
            pltpu.async_copy(ie_v, accn.at[uloc_v.at[0]], ssem, add=True),
            pltpu.async_copy(p_v, accd.at[uloc_v.at[0]], ssem, add=True),
        ]
        for d in descs:
            d.wait()
        return 0

    lax.fori_loop(0, nch, chunk, 0)
    _writeout(accn, accd, num_hbm, den_hbm, ie_v, p_v, cid, sid, _CHU)


_QP = 1024


def _make_count(sl):
    @functools.partial(
        pl.kernel,
        out_type=jax.ShapeDtypeStruct((512,), jnp.int32),
        mesh=_mesh,
        scratch_types=[
            pltpu.VMEM((_QP,), jnp.int32),
            pltpu.VMEM((16,), jnp.int32),
        ],
        compiler_params=pltpu.CompilerParams(
            needs_layout_passes=False, use_tc_tiling_on_sc=False),
    )
    def k(dst_hbm, cnt_hbm, dst_v, out_v):
        cid = lax.axis_index("c")
        sid = lax.axis_index("s")
        wid = cid * 16 + sid
        lo = cid * _HALF

        def cb(i, tot):
            eo = pl.multiple_of(sid * sl + i * _QP, _QP)
            pltpu.sync_copy(dst_hbm.at[pl.ds(eo, _QP)], dst_v)

            def g(gi, t):
                d = dst_v[pl.ds(gi * 16, 16)]
                msk = jnp.logical_and(d >= lo, d < lo + _HALF)
                return t + plsc.all_reduce_population_count(msk)

            return lax.fori_loop(0, _QP // 16, g, tot)

        tot = lax.fori_loop(0, sl // _QP, cb, jnp.zeros((16,), jnp.int32))
        out_v[pl.ds(0, 16)] = tot
        pltpu.sync_copy(out_v,
                        cnt_hbm.at[pl.ds(pl.multiple_of(wid * 16, 16), 16)])

    return k


def _glue(cnt):
    counts = cnt.reshape(32, 16)[:, 0]
    c0, c1 = counts[:16], counts[16:]
    n0 = jnp.sum(c0)
    n1 = jnp.sum(c1)
    q16 = 16 * _Q
    pt0 = ((n0 + q16 - 1) // q16) * _Q
    pt1 = ((n1 + q16 - 1) // q16) * _Q
    b1 = 16 * pt0
    off = jnp.concatenate([jnp.cumsum(c0) - c0, b1 + jnp.cumsum(c1) - c1])
    a32 = jnp.arange(32, dtype=jnp.int32)
    ptc = jnp.where(a32 < 16, pt0, pt1)
    basec = jnp.where(a32 < 16, 0, b1)
    z32 = jnp.zeros((32,), jnp.int32)
    tbl = jnp.stack([off, ptc, basec] + [z32] * 13, axis=1).reshape(-1)
    meta = jnp.stack([pt0, pt1, b1] + [jnp.int32(0)] * 13)
    return meta.astype(jnp.int32), tbl.astype(jnp.int32)


def _make_place(sl, L, kind):
    npay = 3 if kind == "kg" else 4
    nj = _QP // 128

    @functools.partial(
        pl.kernel,
        out_type=[jax.ShapeDtypeStruct((L,), jnp.int32)] * npay,
        mesh=_mesh,
        scratch_types=[
            pltpu.VMEM((_QP,), jnp.int32),
            pltpu.VMEM((_QP,), jnp.int32),
            pltpu.VMEM((_QP,), jnp.int32),
            pltpu.VMEM((_QP,), jnp.int32),
            pltpu.VMEM((_QP,), jnp.int32),
            pltpu.VMEM((nj, 128), jnp.int32),
            pltpu.VMEM((16,), jnp.int32),
            pltpu.SemaphoreType.DMA,
            pltpu.SemaphoreType.DMA,
            pltpu.SemaphoreType.DMA,
        ],
        compiler_params=pltpu.CompilerParams(
            needs_layout_passes=False, use_tc_tiling_on_sc=False),
    )
    def k(dst_hbm, a_hbm, b_hbm, tbl_hbm, *rest):
        outs = rest[:npay]
        (dst_v, a_v, b_v, st1, st2, slots_v, off_v,
         fsem, gsem, ssem) = rest[npay:]
        cid = lax.axis_index("c")
        sid = lax.axis_index("s")
        wid = cid * 16 + sid
        lo = cid * _HALF
        iota = _IOTA()
        pltpu.sync_copy(tbl_hbm.at[pl.ds(pl.multiple_of(wid * 16, 16), 16)],
                        off_v)
        m = off_v[pl.ds(0, 16)]
        off0, pt, base = m[0], m[1], m[2]

        def sb(g, _):
            dst_v[pl.ds(g * 16, 16)] = jnp.zeros((16,), jnp.int32)
            a_v[pl.ds(g * 16, 16)] = jnp.full((16,), _DUMP, jnp.int32)
            return 0

        lax.fori_loop(0, _QP // 16, sb, 0)
        sent = [dst_v] * (npay - 1) + [a_v]

        def fill_issue(o, n):
            for p in range(npay):
                pltpu.async_copy(sent[p].at[pl.ds(0, n)],
                                 outs[p].at[pl.ds(o, n)], fsem)

        def fill_drain(n):
            for p in range(npay):
                pltpu.make_async_copy(outs[p].at[pl.ds(0, n)],
                                      b_v.at[pl.ds(0, n)], fsem).wait()

        nf = pt // _QP

        def fb(i, _):
            o = pl.multiple_of(base + sid * pt + i * _QP, _Q)
            fill_issue(o, _QP)

            @pl.when(i > 0)
            def _():
                fill_drain(_QP)

            return 0

        lax.fori_loop(0, nf, fb, 0)

        @pl.when(nf > 0)
        def _():
            fill_drain(_QP)

        @pl.when(pt - nf * _QP > 0)
        def _():
            o = pl.multiple_of(base + sid * pt + nf * _QP, _Q)
            fill_issue(o, _Q)
            fill_drain(_Q)

        plsc.subcore_barrier()

        trash = L - 32 * _TR + wid * _TR

        def data_list():
            if kind == "kg":
                return [a_v, st1, st2]
            return [a_v, dst_v, b_v, st1]

        def scb(i, off):
            @pl.when(i > 0)
            def _():
                for p in range(npay):
                    for j in range(nj):
                        pltpu.make_async_copy(outs[p].at[pl.ds(0, 128)],
                                              slots_v.at[0], ssem).wait()

            eo = pl.multiple_of(sid * sl + i * _QP, _QP)
            d1 = pltpu.async_copy(dst_hbm.at[pl.ds(eo, _QP)], dst_v, gsem)
            d2 = pltpu.async_copy(a_hbm.at[pl.ds(eo, _QP)], a_v, gsem)
            d3 = pltpu.async_copy(b_hbm.at[pl.ds(eo, _QP)], b_v, gsem)
            d1.wait()
            d2.wait()
            d3.wait()

            def g_body(g, off):
                d16 = dst_v[pl.ds(g * 16, 16)]
                msk = jnp.logical_and(d16 >= lo, d16 < lo + _HALF)
                r = plsc.cumsum(msk.astype(jnp.int32))
                pop = r[15]
                tpos = jnp.bitwise_and(iota + g * 16, _TR - 1)
                slot = jnp.where(msk, off + r - 1, trash + tpos)
                slots_v[g // 8, pl.ds((g % 8) * 16, 16)] = slot
                if kind == "kg":
                    a16 = a_v[pl.ds(g * 16, 16)]
                    b16 = b_v[pl.ds(g * 16, 16)]
                    st1[pl.ds(g * 16, 16)] = a16 * _NREL + b16 - 1
                    st2[pl.ds(g * 16, 16)] = d16 - lo
                else:
                    st1[pl.ds(g * 16, 16)] = d16 - lo
                return off + pop

            off = lax.fori_loop(0, _QP // 16, g_body, off)
            dl = data_list()
            for p in range(npay):
                for j in range(nj):
                    pltpu.async_copy(dl[p].at[pl.ds(j * 128, 128)],
                                     outs[p].at[slots_v.at[j]], ssem)
            return off

        lax.fori_loop(0, sl // _QP, scb, off0)
        for p in range(npay):
            for j in range(nj):
                pltpu.make_async_copy(outs[p].at[pl.ds(0, 128)],
                                      slots_v.at[0], ssem).wait()

    return k


_count_kg = _make_count(_SLK)
_count_u = _make_count(_SLU)
_place_kg = _make_place(_SLK, _LK, "kg")
_place_u = _make_place(_SLU, _LU, "user")


def kernel(user_emb, entity_emb, interact_emb, relation_emb, edge_index,
           edge_type, interact_user_index, interact_item_index, interact_type):
    f32 = jnp.float32
    head = edge_index[0].astype(jnp.int32)
    tail = edge_index[1].astype(jnp.int32)
    etype = edge_type.astype(jnp.int32)
    uidx = interact_user_index.astype(jnp.int32)
    iidx = interact_item_index.astype(jnp.int32)
    titype = interact_type.astype(jnp.int32)

    def pad_to(x, n, v):
        return jnp.concatenate(
            [x, jnp.full((n - x.shape[0],), v, jnp.int32)])

    head_pad = pad_to(head, 16 * _SLK, 1 << 29)
    tail_pad = pad_to(tail, 16 * _SLK, 0)
    etype_pad = pad_to(etype, 16 * _SLK, 1)
    usr_pad = pad_to(uidx, 16 * _SLU, 1 << 29)
    item_pad = pad_to(iidx, 16 * _SLU, 0)
    typ_pad = pad_to(titype, 16 * _SLU, 0)

    meta_k, tbl_k = _glue(_count_kg(head_pad))
    meta_u, tbl_u = _glue(_count_u(usr_pad))
    tail_p, widx_p, hloc_p = _place_kg(head_pad, tail_pad, etype_pad, tbl_k)
    item_p, usr_p, typ_p, uloc_p = _place_u(usr_pad, item_pad, typ_pad,
                                            tbl_u)

    z2 = jnp.zeros((_CHK, _D), f32)
    z1 = jnp.zeros((_CHK,), f32)
    relt = relation_emb.T.astype(f32) * 0.125
    te_flat = interact_emb.reshape(-1).astype(f32)

    e = entity_emb
    u = user_emb
    eres = entity_emb
    ures = user_emb
    w = _w0(e, relt)
    for _ in range(3):
        num_e, den_e = _sc_kg(e, w.reshape(-1), tail_p, widx_p, hloc_p,
                              meta_k, z2, z1)
        num_u, den_u = _sc_user(e, u, te_flat, item_p, usr_p, typ_p, uloc_p,
                                meta_u, z2, z1)
        e, u, eres, ures, w = _finish(num_e, den_e, num_u, den_u,
                                      eres, ures, relt)
    return (eres, ures)


# speedup vs baseline: 11.4758x; 1.9531x over previous
import functools

import jax
import jax.numpy as jnp
from jax import lax
from jax.experimental import pallas as pl
from jax.experimental.pallas import tpu as pltpu
from jax.experimental.pallas import tpu_sc as plsc

_NU = 50000
_NE = 50000
_D = 64
_NREL = 32
_HALF = 25000
_TPS = 1568
_ACC_ROWS = 25088
_DUMP = 25000
_CHK = 256
_CHU = 128
_Q = 512
_NK = 800000
_NNZ = 1000000
_SLK = 50176
_SLU = 63488
_LK = _NK + 32 * _Q + 16 * _Q * 2
_LU = _NNZ + 32 * _Q + 16 * _Q * 2
_FBLK = 2000


def _segs(total, ch):
    segs, o = [], 0
    while o < total:
        n = min(ch, total - o)
        segs.append((o, n))
        o += n
    return tuple(segs)


def _norm_rows(num, den):
    agg = num / (den + 1e-16)
    n = jnp.sqrt(jnp.sum(agg * agg, axis=1, keepdims=True))
    return agg / jnp.maximum(n, 1e-12)


def _w0_body(e_ref, relt_ref, w_ref):
    w_ref[...] = jnp.dot(e_ref[...], relt_ref[...],
                         preferred_element_type=jnp.float32)


@jax.jit
def _w0(e, relt):
    grid = (_NE // _FBLK,)
    return pl.pallas_call(
        _w0_body,
        grid=grid,
        in_specs=[pl.BlockSpec((_FBLK, _D), lambda i: (i, 0)),
                  pl.BlockSpec((_D, _NREL), lambda i: (0, 0))],
        out_specs=pl.BlockSpec((_FBLK, _NREL), lambda i: (i, 0)),
        out_shape=jax.ShapeDtypeStruct((_NE, _NREL), jnp.float32),
    )(e, relt)


def _finish_body(num_e_ref, den_e_ref, num_u_ref, den_u_ref,
                 eres_ref, ures_ref, relt_ref,
                 enew_ref, unew_ref, eout_ref, uout_ref, w_ref):
    e = _norm_rows(num_e_ref[...], den_e_ref[...])
    u = _norm_rows(num_u_ref[...], den_u_ref[...])
    enew_ref[...] = e
    unew_ref[...] = u
    eout_ref[...] = eres_ref[...] + e
    uout_ref[...] = ures_ref[...] + u
    w_ref[...] = jnp.dot(e, relt_ref[...], preferred_element_type=jnp.float32)


@jax.jit
def _finish(num_e, den_e, num_u, den_u, eres, ures, relt):
    grid = (_NE // _FBLK,)
    row_spec = pl.BlockSpec((_FBLK, _D), lambda i: (i, 0))
    col_spec = pl.BlockSpec((_FBLK, 1), lambda i: (i, 0))
    return pl.pallas_call(
        _finish_body,
        grid=grid,
        in_specs=[row_spec, col_spec, row_spec, col_spec, row_spec, row_spec,
                  pl.BlockSpec((_D, _NREL), lambda i: (0, 0))],
        out_specs=[row_spec, row_spec, row_spec, row_spec,
                   pl.BlockSpec((_FBLK, _NREL), lambda i: (i, 0))],
        out_shape=[jax.ShapeDtypeStruct((_NE, _D), jnp.float32)] * 4
        + [jax.ShapeDtypeStruct((_NE, _NREL), jnp.float32)],
    )(num_e, den_e[:, None], num_u, den_u[:, None], eres, ures, relt)


_mesh = plsc.VectorSubcoreMesh(core_axis_name="c", subcore_axis_name="s")
_IOTA = functools.partial(lax.broadcasted_iota, jnp.int32, (16,), 0)


def _tile_ranges(meta_v, cid, sid, ch):
    m = meta_v[pl.ds(0, 16)]
    pt_edges = jnp.where(cid == 0, m[0], m[1])
    base_edge = jnp.where(cid == 0, 0, m[2])
    start_edge = pl.multiple_of(base_edge + sid * pt_edges, ch)
    nch = pt_edges // ch
    return start_edge, nch


def _zero_acc(accn, accd, z2_hbm, z1_hbm, rows_v, p_v, sid, ch):
    pltpu.sync_copy(z2_hbm.at[pl.ds(0, ch)], rows_v)
    pltpu.sync_copy(z1_hbm.at[pl.ds(0, ch)], p_v)
    row0 = sid * _TPS
    for o, n in _segs(_TPS, ch):
        pltpu.sync_copy(rows_v.at[pl.ds(0, n)],
                        accn.at[pl.ds(row0 + o, n)])
        pltpu.sync_copy(p_v.at[pl.ds(0, n)],
                        accd.at[pl.ds(row0 + o, n)])
    plsc.subcore_barrier()


def _writeout(accn, accd, num_hbm, den_hbm, rows_v, p_v, cid, sid, ch):
    plsc.subcore_barrier()
    row0 = sid * _TPS
    gbase = cid * _HALF + row0

    def flush(segs):
        for o, n in segs:
            pltpu.sync_copy(accn.at[pl.ds(row0 + o, n)],
                            rows_v.at[pl.ds(0, n)])
            pltpu.sync_copy(rows_v.at[pl.ds(0, n)],
                            num_hbm.at[pl.ds(gbase + o, n)])
            pltpu.sync_copy(accd.at[pl.ds(row0 + o, n)],
                            p_v.at[pl.ds(0, n)])
            pltpu.sync_copy(p_v.at[pl.ds(0, n)],
                            den_hbm.at[pl.ds(gbase + o, n)])

    @pl.when(sid < 15)
    def _():
        flush(_segs(_TPS, ch))

    @pl.when(sid == 15)
    def _():
        flush(_segs(_HALF - 15 * _TPS, ch))


@functools.partial(
    pl.kernel,
    out_type=[jax.ShapeDtypeStruct((_NE, _D), jnp.float32),
              jax.ShapeDtypeStruct((_NE,), jnp.float32)],
    mesh=_mesh,
    scratch_types=[
        pltpu.VMEM_SHARED((_ACC_ROWS, _D), jnp.float32),
        pltpu.VMEM_SHARED((_ACC_ROWS,), jnp.float32),
        pltpu.VMEM((_CHK,), jnp.int32),
        pltpu.VMEM((_CHK,), jnp.int32),
        pltpu.VMEM((2, 128), jnp.int32),
        pltpu.VMEM((_CHK, _D), jnp.float32),
        pltpu.VMEM((_CHK,), jnp.float32),
        pltpu.VMEM((_CHK,), jnp.float32),
        pltpu.VMEM((16,), jnp.int32),
        pltpu.SemaphoreType.DMA,
        pltpu.SemaphoreType.DMA,
    ],
    compiler_params=pltpu.CompilerParams(
        needs_layout_passes=False, use_tc_tiling_on_sc=False),
)
def _sc_kg(ent_hbm, wflat_hbm, tail_hbm, widx_hbm, hloc_hbm, meta_hbm,
           z2_hbm, z1_hbm, num_hbm, den_hbm,
           accn, accd, tidx_v, widx_v, hloc_v, rows_v, w_v, p_v, meta_v,
           gsem, ssem):
    cid = lax.axis_index("c")
    sid = lax.axis_index("s")
    pltpu.sync_copy(meta_hbm, meta_v)
    start_edge, nch = _tile_ranges(meta_v, cid, sid, _CHK)
    _zero_acc(accn, accd, z2_hbm, z1_hbm, rows_v, p_v, sid, _CHK)

    iota = _IOTA()

    def chunk(i, _):
        eoff = pl.multiple_of(start_edge + i * _CHK, _CHK)
        pltpu.sync_copy(tail_hbm.at[pl.ds(eoff, _CHK)], tidx_v)
        pltpu.sync_copy(widx_hbm.at[pl.ds(eoff, _CHK)], widx_v)
        for j in range(2):
            pltpu.sync_copy(hloc_hbm.at[pl.ds(eoff + j * 128, 128)],
                            hloc_v.at[j])
        descs = []
        for j in range(2):
            descs.append(pltpu.async_copy(
                ent_hbm.at[tidx_v.at[pl.ds(j * 128, 128)]],
                rows_v.at[pl.ds(j * 128, 128)], gsem))
            descs.append(pltpu.async_copy(
                wflat_hbm.at[widx_v.at[pl.ds(j * 128, 128)]],
                w_v.at[pl.ds(j * 128, 128)], gsem))
        for d in descs:
            d.wait()

        def g_body(g, _):
            w16 = w_v[pl.ds(g * 16, 16)]
            p16 = jnp.exp(w16)
            p_v[pl.ds(g * 16, 16)] = p16
            rbase = iota + g * 16

            def c_body(c8, _):
                for k in range(8):
                    c = c8 * 8 + k
                    col = jnp.full((16,), c, jnp.int32)
                    v = plsc.load_gather(rows_v, [rbase, col])
                    plsc.store_scatter(rows_v, [rbase, col], v * p16)
                return 0

            lax.fori_loop(0, 8, c_body, 0)
            return 0

        lax.fori_loop(0, _CHK // 16, g_body, 0)

        descs = []
        for j in range(2):
            descs.append(pltpu.async_copy(
                rows_v.at[pl.ds(j * 128, 128)],
                accn.at[hloc_v.at[j]], ssem, add=True))
            descs.append(pltpu.async_copy(
                p_v.at[pl.ds(j * 128, 128)],
                accd.at[hloc_v.at[j]], ssem, add=True))
        for d in descs:
            d.wait()
        return 0

    lax.fori_loop(0, nch, chunk, 0)
    _writeout(accn, accd, num_hbm, den_hbm, rows_v, p_v, cid, sid, _CHK)


@functools.partial(
    pl.kernel,
    out_type=[jax.ShapeDtypeStruct((_NU, _D), jnp.float32),
              jax.ShapeDtypeStruct((_NU,), jnp.float32)],
    mesh=_mesh,
    scratch_types=[
        pltpu.VMEM_SHARED((_ACC_ROWS, _D), jnp.float32),
        pltpu.VMEM_SHARED((_ACC_ROWS,), jnp.float32),
        pltpu.VMEM((_CHU,), jnp.int32),
        pltpu.VMEM((_CHU,), jnp.int32),
        pltpu.VMEM((1, 128), jnp.int32),
        pltpu.VMEM((_CHU, _D), jnp.float32),
        pltpu.VMEM((_CHU, _D), jnp.float32),
        pltpu.VMEM((_CHU,), jnp.int32),
        pltpu.VMEM((_CHU,), jnp.float32),
        pltpu.VMEM((256,), jnp.float32),
        pltpu.VMEM((16,), jnp.int32),
        pltpu.SemaphoreType.DMA,
        pltpu.SemaphoreType.DMA,
    ],
    compiler_params=pltpu.CompilerParams(
        needs_layout_passes=False, use_tc_tiling_on_sc=False),
)
def _sc_user(ent_hbm, utbl_hbm, te_hbm, item_hbm, usr_hbm, typ_hbm, uloc_hbm,
             meta_hbm, z2_hbm, z1_hbm, num_hbm, den_hbm,
             accn, accd, iidx_v, uidx_v, uloc_v, ie_v, ue_v, typ_v, p_v,
             te_v, meta_v, gsem, ssem):
    cid = lax.axis_index("c")
    sid = lax.axis_index("s")
    pltpu.sync_copy(meta_hbm, meta_v)
    pltpu.sync_copy(te_hbm, te_v)
    start_edge, nch = _tile_ranges(meta_v, cid, sid, _CHU)
    _zero_acc(accn, accd, z2_hbm, z1_hbm, ie_v, p_v, sid, _CHU)

    iota = _IOTA()

    def chunk(i, _):
        eoff = pl.multiple_of(start_edge + i * _CHU, _CHU)
        pltpu.sync_copy(item_hbm.at[pl.ds(eoff, _CHU)], iidx_v)
        pltpu.sync_copy(usr_hbm.at[pl.ds(eoff, _CHU)], uidx_v)
        pltpu.sync_copy(typ_hbm.at[pl.ds(eoff, _CHU)], typ_v)
        pltpu.sync_copy(uloc_hbm.at[pl.ds(eoff, 128)], uloc_v.at[0])
        descs = [
            pltpu.async_copy(ent_hbm.at[iidx_v], ie_v, gsem),
            pltpu.async_copy(utbl_hbm.at[uidx_v], ue_v, gsem),
        ]
        for d in descs:
            d.wait()

        def g_body(g, _):
            rbase = iota + g * 16
            tb = typ_v[pl.ds(g * 16, 16)] * 64

            def att_body(c8, att):
                for k in range(8):
                    c = c8 * 8 + k
                    col = jnp.full((16,), c, jnp.int32)
                    ue = plsc.load_gather(ue_v, [rbase, col])
                    ie = plsc.load_gather(ie_v, [rbase, col])
                    te = plsc.load_gather(te_v, [tb + c])
                    att = att + ue * te * ie
                return att

            att = lax.fori_loop(0, 8, att_body,
                                jnp.zeros((16,), jnp.float32))
            p16 = jnp.exp(att)
            p_v[pl.ds(g * 16, 16)] = p16

            def sc_body(c8, _):
                for k in range(8):
                    c = c8 * 8 + k
                    col = jnp.full((16,), c, jnp.int32)
                    ie = plsc.load_gather(ie_v, [rbase, col])
                    plsc.store_scatter(ie_v, [rbase, col], ie * p16)
                return 0

            lax.fori_loop(0, 8, sc_body, 0)
            return 0

        lax.fori_loop(0, _CHU // 16, g_body, 0)

        descs = [
            pltpu.async_copy(ie_v, accn.at[uloc_v.at[0]], ssem, add=True),
            pltpu.async_copy(p_v, accd.at[uloc_v.at[0]], ssem, add=True),
        ]
        for d in descs:
            d.wait()
        return 0

    lax.fori_loop(0, nch, chunk, 0)
    _writeout(accn, accd, num_hbm, den_hbm, ie_v, p_v, cid, sid, _CHU)


_QP = 1024


def _make_count(sl):
    @functools.partial(
        pl.kernel,
        out_type=jax.ShapeDtypeStruct((512,), jnp.int32),
        mesh=_mesh,
        scratch_types=[
            pltpu.VMEM((_QP,), jnp.int32),
            pltpu.VMEM((16,), jnp.int32),
        ],
        compiler_params=pltpu.CompilerParams(
            needs_layout_passes=False, use_tc_tiling_on_sc=False),
    )
    def k(dst_hbm, cnt_hbm, dst_v, out_v):
        cid = lax.axis_index("c")
        sid = lax.axis_index("s")
        wid = cid * 16 + sid
        lo = cid * _HALF

        def cb(i, tot):
            eo = pl.multiple_of(sid * sl + i * _QP, _QP)
            pltpu.sync_copy(dst_hbm.at[pl.ds(eo, _QP)], dst_v)

            def g(gi, t):
                d = dst_v[pl.ds(gi * 16, 16)]
                msk = jnp.logical_and(d >= lo, d < lo + _HALF)
                return t + plsc.all_reduce_population_count(msk)

            return lax.fori_loop(0, _QP // 16, g, tot)

        tot = lax.fori_loop(0, sl // _QP, cb, jnp.zeros((16,), jnp.int32))
        out_v[pl.ds(0, 16)] = tot
        pltpu.sync_copy(out_v,
                        cnt_hbm.at[pl.ds(pl.multiple_of(wid * 16, 16), 16)])

    return k


def _glue(cnt):
    counts = cnt.reshape(32, 16)[:, 0]
    rc = ((counts + _Q - 1) // _Q) * _Q
    c0, c1 = rc[:16], rc[16:]
    n0 = jnp.sum(c0)
    n1 = jnp.sum(c1)
    q16 = 16 * _Q
    pt0 = ((n0 + q16 - 1) // q16) * _Q
    pt1 = ((n1 + q16 - 1) // q16) * _Q
    b1 = 16 * pt0
    off = jnp.concatenate([jnp.cumsum(c0) - c0, b1 + jnp.cumsum(c1) - c1])
    a32 = jnp.arange(32, dtype=jnp.int32)
    ptc = jnp.where(a32 < 16, pt0, pt1)
    basec = jnp.where(a32 < 16, 0, b1)
    z32 = jnp.zeros((32,), jnp.int32)
    tbl = jnp.stack([off, ptc, basec] + [z32] * 13, axis=1).reshape(-1)
    meta = jnp.stack([pt0, pt1, b1] + [jnp.int32(0)] * 13)
    return meta.astype(jnp.int32), tbl.astype(jnp.int32)


def _make_place(sl, L, kind):
    npay = 3 if kind == "kg" else 4
    nj = _QP // 128

    @functools.partial(
        pl.kernel,
        out_type=[jax.ShapeDtypeStruct((L,), jnp.int32)] * npay,
        mesh=_mesh,
        scratch_types=[
            pltpu.VMEM((_QP,), jnp.int32),
            pltpu.VMEM((_QP,), jnp.int32),
            pltpu.VMEM((_QP,), jnp.int32),
            pltpu.VMEM((1600,), jnp.int32),
            pltpu.VMEM((1600,), jnp.int32),
            pltpu.VMEM((1600,), jnp.int32),
            pltpu.VMEM((1600,), jnp.int32),
            pltpu.VMEM((16,), jnp.int32),
            pltpu.SemaphoreType.DMA,
            pltpu.SemaphoreType.DMA,
        ],
        compiler_params=pltpu.CompilerParams(
            needs_layout_passes=False, use_tc_tiling_on_sc=False),
    )
    def k(dst_hbm, a_hbm, b_hbm, tbl_hbm, *rest):
        outs = rest[:npay]
        (dst_v, a_v, b_v, sg1, sg2, sg3, sg4, off_v,
         fsem, gsem) = rest[npay:]
        stgs = [sg1, sg2, sg3, sg4][:npay]
        cid = lax.axis_index("c")
        sid = lax.axis_index("s")
        wid = cid * 16 + sid
        lo = cid * _HALF
        iota = _IOTA()
        pltpu.sync_copy(tbl_hbm.at[pl.ds(pl.multiple_of(wid * 16, 16), 16)],
                        off_v)
        m = off_v[pl.ds(0, 16)]
        off0, pt, base = m[0], m[1], m[2]

        def sb(g, _):
            dst_v[pl.ds(g * 16, 16)] = jnp.zeros((16,), jnp.int32)
            a_v[pl.ds(g * 16, 16)] = jnp.full((16,), _DUMP, jnp.int32)
            return 0

        lax.fori_loop(0, _QP // 16, sb, 0)
        sent = [dst_v] * (npay - 1) + [a_v]

        def fill_issue(o, n):
            for p in range(npay):
                pltpu.async_copy(sent[p].at[pl.ds(0, n)],
                                 outs[p].at[pl.ds(o, n)], fsem)

        def fill_drain(n):
            for p in range(npay):
                pltpu.make_async_copy(outs[p].at[pl.ds(0, n)],
                                      b_v.at[pl.ds(0, n)], fsem).wait()

        nf = pt // _QP

        def fb(i, _):
            o = pl.multiple_of(base + sid * pt + i * _QP, _Q)
            fill_issue(o, _QP)

            @pl.when(i > 0)
            def _():
                fill_drain(_QP)

            return 0

        lax.fori_loop(0, nf, fb, 0)

        @pl.when(nf > 0)
        def _():
            fill_drain(_QP)

        @pl.when(pt - nf * _QP > 0)
        def _():
            o = pl.multiple_of(base + sid * pt + nf * _QP, _Q)
            fill_issue(o, _Q)
            fill_drain(_Q)

        plsc.subcore_barrier()

        sent_vals = ([0, 0, _DUMP] if kind == "kg" else [0, 0, 0, _DUMP])

        def flush_block(src, dst_off):
            for p in range(npay):
                pltpu.sync_copy(
                    stgs[p].at[pl.ds(src, _Q)],
                    outs[p].at[pl.ds(pl.multiple_of(dst_off, _Q), _Q)])

        def scb(i, carry):
            fill, flushed = carry
            eo = pl.multiple_of(sid * sl + i * _QP, _QP)
            d1 = pltpu.async_copy(dst_hbm.at[pl.ds(eo, _QP)], dst_v, gsem)
            d2 = pltpu.async_copy(a_hbm.at[pl.ds(eo, _QP)], a_v, gsem)
            d3 = pltpu.async_copy(b_hbm.at[pl.ds(eo, _QP)], b_v, gsem)
            d1.wait()
            d2.wait()
            d3.wait()

            def g_body(g, fill):
                d16 = dst_v[pl.ds(g * 16, 16)]
                msk = jnp.logical_and(d16 >= lo, d16 < lo + _HALF)
                pop = plsc.all_reduce_population_count(msk)[0]
                a16 = a_v[pl.ds(g * 16, 16)]
                if kind == "kg":
                    b16 = b_v[pl.ds(g * 16, 16)]
                    vals = [a16, a16 * _NREL + b16 - 1, d16 - lo]
                else:
                    vals = [a16, d16, b_v[pl.ds(g * 16, 16)], d16 - lo]
                for p in range(npay):
                    plsc.store_compressed(stgs[p].at[pl.ds(fill, 16)],
                                          vals[p], mask=msk)
                return fill + pop

            fill = lax.fori_loop(0, _QP // 16, g_body, fill)
            nb = fill // _Q

            @pl.when(nb >= 1)
            def _():
                flush_block(0, off0 + flushed)

            @pl.when(nb >= 2)
            def _():
                flush_block(_Q, off0 + flushed + _Q)

            @pl.when(nb >= 3)
            def _():
                flush_block(2 * _Q, off0 + flushed + 2 * _Q)

            rem = fill - nb * _Q

            def shift(j, _):
                for p in range(npay):
                    v = stgs[p][pl.ds(nb * _Q + j * 16, 16)]
                    stgs[p][pl.ds(j * 16, 16)] = v
                return 0

            @pl.when(nb >= 1)
            def _():
                lax.fori_loop(0, (rem + 15) // 16, shift, 0)

            return (rem, flushed + nb * _Q)

        rem, flushed = lax.fori_loop(0, sl // _QP, scb, (off0 * 0, off0 * 0))

        @pl.when(rem > 0)
        def _():
            def sp(j, _):
                for p in range(npay):
                    stgs[p][pl.ds(rem + j * 16, 16)] = jnp.full(
                        (16,), sent_vals[p], jnp.int32)
                return 0

            lax.fori_loop(0, (_Q - rem + 15) // 16, sp, 0)
            flush_block(0, off0 + flushed)

    return k


_count_kg = _make_count(_SLK)
_count_u = _make_count(_SLU)
_place_kg = _make_place(_SLK, _LK, "kg")
_place_u = _make_place(_SLU, _LU, "user")


def kernel(user_emb, entity_emb, interact_emb, relation_emb, edge_index,
           edge_type, interact_user_index, interact_item_index, interact_type):
    f32 = jnp.float32
    head = edge_index[0].astype(jnp.int32)
    tail = edge_index[1].astype(jnp.int32)
    etype = edge_type.astype(jnp.int32)
    uidx = interact_user_index.astype(jnp.int32)
    iidx = interact_item_index.astype(jnp.int32)
    titype = interact_type.astype(jnp.int32)

    def pad_to(x, n, v):
        return jnp.concatenate(
            [x, jnp.full((n - x.shape[0],), v, jnp.int32)])

    head_pad = pad_to(head, 16 * _SLK, 1 << 29)
    tail_pad = pad_to(tail, 16 * _SLK, 0)
    etype_pad = pad_to(etype, 16 * _SLK, 1)
    usr_pad = pad_to(uidx, 16 * _SLU, 1 << 29)
    item_pad = pad_to(iidx, 16 * _SLU, 0)
    typ_pad = pad_to(titype, 16 * _SLU, 0)

    meta_k, tbl_k = _glue(_count_kg(head_pad))
    meta_u, tbl_u = _glue(_count_u(usr_pad))
    tail_p, widx_p, hloc_p = _place_kg(head_pad, tail_pad, etype_pad, tbl_k)
    item_p, usr_p, typ_p, uloc_p = _place_u(usr_pad, item_pad, typ_pad,
                                            tbl_u)

    z2 = jnp.zeros((_CHK, _D), f32)
    z1 = jnp.zeros((_CHK,), f32)
    relt = relation_emb.T.astype(f32) * 0.125
    te_flat = interact_emb.reshape(-1).astype(f32)

    e = entity_emb
    u = user_emb
    eres = entity_emb
    ures = user_emb
    w = _w0(e, relt)
    for _ in range(3):
        num_e, den_e = _sc_kg(e, w.reshape(-1), tail_p, widx_p, hloc_p,
                              meta_k, z2, z1)
        num_u, den_u = _sc_user(e, u, te_flat, item_p, usr_p, typ_p, uloc_p,
                                meta_u, z2, z1)
        e, u, eres, ures, w = _finish(num_e, den_e, num_u, den_u,
                                      eres, ures, relt)
    return (eres, ures)
